# baseline (speedup 1.0000x reference)
import functools
import math

import jax
import jax.numpy as jnp
from jax import lax
from jax.experimental import pallas as pl
from jax.experimental.pallas import tpu as pltpu
from jax.experimental.pallas import tpu_sc as plsc

NC = 2
NS = 16
LANES = 16
EK = 112


NBUF = 3


@functools.cache
def _make_spmm(n_nodes: int, feat: int, e_pad: int, subch: int):
    ekw = EK // LANES
    eks = EK * subch
    edges_per_sub = e_pad // NS
    n_slots = edges_per_sub // eks
    assert edges_per_sub % eks == 0 and n_slots % NBUF == 0 and n_slots >= 6
    rmain = ((n_nodes // NS) + 7) // 8 * 8
    rlast = n_nodes - (NS - 1) * rmain
    assert 0 < rlast <= rmain and rlast % 8 == 0
    zr = 8
    nslice = feat // LANES
    ec = 3 * EK
    ecs = ec * subch

    mesh = plsc.VectorSubcoreMesh(core_axis_name="c", subcore_axis_name="s")

    @functools.partial(
        pl.kernel,
        mesh=mesh,
        compiler_params=pltpu.CompilerParams(
            use_tc_tiling_on_sc=(feat % 128 == 0)),
        out_type=jax.ShapeDtypeStruct((2 * n_nodes, feat), jnp.float32),
        scratch_types=(
            [pltpu.VMEM((ecs,), jnp.int32)] * NBUF
            + [pltpu.VMEM((subch, EK), jnp.int32)] * NBUF
            + [pltpu.VMEM((subch, EK), jnp.int32)] * NBUF
            + [pltpu.VMEM((eks, feat), jnp.float32)] * NBUF
            + [pltpu.VMEM((zr, feat), jnp.float32)]
            + [pltpu.VMEM_SHARED((n_nodes, feat), jnp.float32)]
            + [pltpu.SemaphoreType.DMA] * (3 * NBUF)
        ),
    )
    def spmm(sup, edata, out, *refs):
        ed = refs[0:NBUF]
        srcv = refs[NBUF:2 * NBUF]
        dstv = refs[2 * NBUF:3 * NBUF]
        rows = refs[3 * NBUF:4 * NBUF]
        zblk = refs[4 * NBUF]
        hacc = refs[4 * NBUF + 1]
        gsem = refs[4 * NBUF + 2:4 * NBUF + 2 + NBUF]
        ssem = refs[4 * NBUF + 2 + NBUF:4 * NBUF + 2 + 2 * NBUF]
        esem = refs[4 * NBUF + 2 + 2 * NBUF:]
        c = lax.axis_index("c")
        s = lax.axis_index("s")

        zeros16 = jnp.zeros((LANES,), jnp.float32)

        def zfill(i, carry):
            for j in range(nslice):
                zblk[i, pl.ds(LANES * j, LANES)] = zeros16
            return carry

        lax.fori_loop(0, zr, zfill, 0)
        row0 = s * rmain
        nrows = jnp.where(s == NS - 1, rlast, rmain)

        def zcopy(t, carry):
            pltpu.sync_copy(zblk, hacc.at[pl.ds(row0 + t * zr, zr)])
            return carry

        lax.fori_loop(0, nrows // zr, zcopy, 0)
        plsc.subcore_barrier()

        ebase = s * n_slots * ecs
        coff = jnp.full((LANES,), c * n_nodes, jnp.int32)
        gdn = lax.GatherDimensionNumbers(
            offset_dims=(), collapsed_slice_dims=(0,), start_index_map=(0,))

        def fire_ed(i, b):
            pltpu.async_copy(
                edata.at[pl.ds(ebase + i * ecs, ecs)], ed[b], esem[b])

        def drain_ed(i, b):
            pltpu.make_async_copy(
                edata.at[pl.ds(ebase + i * ecs, ecs)], ed[b], esem[b]).wait()

        def gather_pairs(b):
            return [(sup.at[srcv[b].at[k]], rows[b].at[pl.ds(k * EK, EK)])
                    for k in range(subch)]

        def scatter_pairs(b):
            return [(rows[b].at[pl.ds(k * EK, EK)], hacc.at[dstv[b].at[k]])
                    for k in range(subch)]

        def fire_gathers(i, b):
            drain_ed(i, b)
            for k in range(subch):
                for j in range(ekw):
                    sl = pl.ds(LANES * j, LANES)
                    srcv[b][k, sl] = ed[b][pl.ds(k * ec + LANES * j, LANES)] + coff
            for s_, d_ in gather_pairs(b):
                pltpu.async_copy(s_, d_, gsem[b])

        def half(i, b):
            nxt = (b + 1) % NBUF
            n2 = (b + 2) % NBUF
            for s_, d_ in gather_pairs(b):
                pltpu.make_async_copy(s_, d_, gsem[b]).wait()

            @pl.when(i >= 2)
            def _():
                for s_, d_ in scatter_pairs(nxt):
                    pltpu.make_async_copy(s_, d_, ssem[nxt]).wait()

            @pl.when(i + 1 < n_slots)
            def _():
                fire_gathers(i + 1, nxt)

            for k in range(subch):
                for j in range(ekw):
                    sl = pl.ds(LANES * j, LANES)
                    dstv[b][k, sl] = ed[b][pl.ds(k * ec + EK + LANES * j, LANES)]

            def scale(g, carry2):
                ch = g // ekw
                woff = ch * ec + 2 * EK + (g - ch * ekw) * LANES
                w16 = lax.bitcast_convert_type(
                    ed[b][pl.ds(woff, LANES)], jnp.float32)
                for k in range(LANES):
                    w = lax.gather(
                        w16, jnp.full((LANES, 1), k, jnp.int32), gdn,
                        slice_sizes=(1,),
                        mode=lax.GatherScatterMode.PROMISE_IN_BOUNDS)
                    e = g * LANES + k
                    for j in range(nslice):
                        sl = pl.ds(LANES * j, LANES)
                        rows[b][e, sl] = rows[b][e, sl] * w
                return carry2

            lax.fori_loop(0, eks // LANES, scale, 0)

            @pl.when(i + 2 < n_slots)
            def _():
                fire_ed(i + 2, n2)

            for s_, d_ in scatter_pairs(b):
                pltpu.async_copy(s_, d_, ssem[b], add=True)

        fire_ed(0, 0)
        fire_gathers(0, 0)
        fire_ed(1, 1)

        def tri(t, carry):
            for k in range(NBUF):
                half(t * NBUF + k, k)
            return carry

        lax.fori_loop(0, n_slots // NBUF, tri, 0)
        for i in (n_slots - 2, n_slots - 1):
            for s_, d_ in scatter_pairs(i % NBUF):
                pltpu.make_async_copy(s_, d_, ssem[i % NBUF]).wait()
        plsc.subcore_barrier()

        @pl.when(s < NS - 1)
        def _():
            pltpu.sync_copy(hacc.at[pl.ds(row0, rmain)],
                            out.at[pl.ds(c * n_nodes + row0, rmain)])

        @pl.when(s == NS - 1)
        def _():
            pltpu.sync_copy(hacc.at[pl.ds(row0, rlast)],
                            out.at[pl.ds(c * n_nodes + row0, rlast)])

    return spmm


def _fused_body(axa_ref, axb_ref, w1_ref, w3_ref, b1_ref, o_ref):
    ax = jnp.concatenate([axa_ref[...], axb_ref[...]], axis=1)
    h = jnp.maximum(
        jnp.dot(ax, w1_ref[...], preferred_element_type=jnp.float32)
        + b1_ref[...], 0.0)
    o_ref[...] = jnp.dot(h, w3_ref[0], preferred_element_type=jnp.float32)


def _tc_fused(axcat, W1, w3, b1r, n, bm):
    _, fx = axcat.shape
    d, hh = W1.shape
    _, _, f2 = w3.shape
    gi = n // bm
    return pl.pallas_call(
        _fused_body,
        grid=(gi, NC),
        in_specs=[
            pl.BlockSpec((bm, fx), lambda i, j: (i, 0)),
            pl.BlockSpec((bm, fx), lambda i, j, _g=gi: (_g + i, 0)),
            pl.BlockSpec((d, hh), lambda i, j: (0, 0)),
            pl.BlockSpec((1, hh, f2), lambda i, j: (j, 0, 0)),
            pl.BlockSpec((1, hh), lambda i, j: (0, 0)),
        ],
        out_specs=pl.BlockSpec((bm, f2), lambda i, j, _g=gi: (j * _g + i, 0)),
        out_shape=jax.ShapeDtypeStruct((NC * n, f2), jnp.float32),
    )(axcat, axcat, W1, w3, b1r)


def _final_body(za_ref, zb_ref, b2_ref, o_ref, *, n_cls):
    z = jnp.concatenate([za_ref[...], zb_ref[...]], axis=1) + b2_ref[...]
    col = lax.broadcasted_iota(jnp.int32, z.shape, 1)
    zm = jnp.where(col < n_cls, z, jnp.float32(-1e30))
    m = jnp.max(zm, axis=1, keepdims=True)
    lse = jnp.log(jnp.sum(jnp.exp(zm - m), axis=1, keepdims=True))
    o_ref[...] = (z - m - lse)[:, :n_cls]


def _tc_final(ocat, b2r, n, n_cls, bm):
    _, f2 = ocat.shape
    cp = b2r.shape[1]
    gi = n // bm
    return pl.pallas_call(
        functools.partial(_final_body, n_cls=n_cls),
        grid=(gi,),
        in_specs=[
            pl.BlockSpec((bm, f2), lambda i: (i, 0)),
            pl.BlockSpec((bm, f2), lambda i, _g=gi: (_g + i, 0)),
            pl.BlockSpec((1, cp), lambda i: (0, 0)),
        ],
        out_specs=pl.BlockSpec((bm, n_cls), lambda i: (i, 0)),
        out_shape=jax.ShapeDtypeStruct((n, n_cls), jnp.float32),
    )(ocat, ocat, b2r)


def kernel(x, edge_index, edge_weight, W1, b1, W2, b2):
    n, d = x.shape
    h = W1.shape[1]
    n_cls = W2.shape[1]
    e = edge_weight.shape[0]
    bm = 1000
    cp = 64

    sub2 = 4
    quantum = NS * EK * NBUF * sub2
    e_pad = math.ceil(e / quantum) * quantum
    if e_pad != e:
        fill = jnp.arange(e_pad - e, dtype=jnp.int32) % n
        src = jnp.concatenate([edge_index[0], fill])
        dst = jnp.concatenate([edge_index[1], fill])
        ew = jnp.concatenate(
            [edge_weight, jnp.zeros((e_pad - e,), jnp.float32)])
    else:
        src, dst, ew = edge_index[0], edge_index[1], edge_weight
    n_chunks = e_pad // (NS * EK)
    edata = jnp.stack([
        src.reshape(NS, n_chunks, EK),
        dst.reshape(NS, n_chunks, EK),
        lax.bitcast_convert_type(ew, jnp.int32).reshape(NS, n_chunks, EK),
    ], axis=2).reshape(-1)

    W2p = jnp.pad(W2, ((0, 0), (0, cp - n_cls)))
    f2 = cp // NC
    w3 = jnp.stack([W2p[:, k * f2:(k + 1) * f2] for k in range(NC)])
    b2r = jnp.pad(b2, (0, cp - n_cls)).reshape(1, cp)
    b1r = b1.reshape(1, h)

    xcat = jnp.concatenate([x[:, :d // NC], x[:, d // NC:]])
    axcat = _make_spmm(n, d // NC, e_pad, 3)(xcat, edata)
    s2cat = _tc_fused(axcat, W1, w3, b1r, n, bm)
    ocat = _make_spmm(n, cp // NC, e_pad, sub2)(s2cat, edata)
    return _tc_final(ocat, b2r, n, n_cls, bm)


# speedup vs baseline: 1.8362x; 1.8362x over previous
import functools
import math

import jax
import jax.numpy as jnp
from jax import lax
from jax.experimental import pallas as pl
from jax.experimental.pallas import tpu as pltpu
from jax.experimental.pallas import tpu_sc as plsc

NC = 2
NS = 16
LANES = 16
EK = 112


NBUF = 3


@functools.cache
def _make_spmm(n_nodes: int, feat: int, e_pad: int, subch: int,
               split: str = "col"):
    ekw = EK // LANES
    eks = EK * subch
    edges_per_sub = e_pad // NS
    n_slots = edges_per_sub // eks
    if split == "edge":
        assert n_slots % NC == 0
        n_slots //= NC
    assert edges_per_sub % eks == 0 and n_slots % NBUF == 0 and n_slots >= 6
    rmain = ((n_nodes // NS) + 7) // 8 * 8
    rlast = n_nodes - (NS - 1) * rmain
    assert 0 < rlast <= rmain and rlast % 8 == 0
    zr = 8
    nslice = feat // LANES
    ec = 3 * EK
    ecs = ec * subch

    mesh = plsc.VectorSubcoreMesh(core_axis_name="c", subcore_axis_name="s")

    @functools.partial(
        pl.kernel,
        mesh=mesh,
        compiler_params=pltpu.CompilerParams(
            use_tc_tiling_on_sc=(feat % 128 == 0)),
        out_type=jax.ShapeDtypeStruct((2 * n_nodes, feat), jnp.float32),
        scratch_types=(
            [pltpu.VMEM((ecs,), jnp.int32)] * NBUF
            + [pltpu.VMEM((subch, EK), jnp.int32)] * NBUF
            + [pltpu.VMEM((subch, EK), jnp.int32)] * NBUF
            + [pltpu.VMEM((eks, feat), jnp.float32)] * NBUF
            + [pltpu.VMEM((zr, feat), jnp.float32)]
            + [pltpu.VMEM_SHARED((n_nodes, feat), jnp.float32)]
            + [pltpu.SemaphoreType.DMA] * (3 * NBUF)
        ),
    )
    def spmm(sup, edata, out, *refs):
        ed = refs[0:NBUF]
        srcv = refs[NBUF:2 * NBUF]
        dstv = refs[2 * NBUF:3 * NBUF]
        rows = refs[3 * NBUF:4 * NBUF]
        zblk = refs[4 * NBUF]
        hacc = refs[4 * NBUF + 1]
        gsem = refs[4 * NBUF + 2:4 * NBUF + 2 + NBUF]
        ssem = refs[4 * NBUF + 2 + NBUF:4 * NBUF + 2 + 2 * NBUF]
        esem = refs[4 * NBUF + 2 + 2 * NBUF:]
        c = lax.axis_index("c")
        s = lax.axis_index("s")

        zeros16 = jnp.zeros((LANES,), jnp.float32)

        def zfill(i, carry):
            for j in range(nslice):
                zblk[i, pl.ds(LANES * j, LANES)] = zeros16
            return carry

        lax.fori_loop(0, zr, zfill, 0)
        row0 = s * rmain
        nrows = jnp.where(s == NS - 1, rlast, rmain)

        def zcopy(t, carry):
            pltpu.sync_copy(zblk, hacc.at[pl.ds(row0 + t * zr, zr)])
            return carry

        lax.fori_loop(0, nrows // zr, zcopy, 0)
        plsc.subcore_barrier()

        if split == "edge":
            ebase = (s * NC + c) * n_slots * ecs
            coff = jnp.full((LANES,), 0, jnp.int32)
        else:
            ebase = s * n_slots * ecs
            coff = jnp.full((LANES,), c * n_nodes, jnp.int32)
        gdn = lax.GatherDimensionNumbers(
            offset_dims=(), collapsed_slice_dims=(0,), start_index_map=(0,))

        def fire_ed(i, b):
            pltpu.async_copy(
                edata.at[pl.ds(ebase + i * ecs, ecs)], ed[b], esem[b])

        def drain_ed(i, b):
            pltpu.make_async_copy(
                edata.at[pl.ds(ebase + i * ecs, ecs)], ed[b], esem[b]).wait()

        def gather_pairs(b):
            return [(sup.at[srcv[b].at[k]], rows[b].at[pl.ds(k * EK, EK)])
                    for k in range(subch)]

        def scatter_pairs(b):
            return [(rows[b].at[pl.ds(k * EK, EK)], hacc.at[dstv[b].at[k]])
                    for k in range(subch)]

        def fire_gathers(i, b):
            drain_ed(i, b)
            for k in range(subch):
                for j in range(ekw):
                    sl = pl.ds(LANES * j, LANES)
                    srcv[b][k, sl] = ed[b][pl.ds(k * ec + LANES * j, LANES)] + coff
            for s_, d_ in gather_pairs(b):
                pltpu.async_copy(s_, d_, gsem[b])

        def half(i, b):
            nxt = (b + 1) % NBUF
            n2 = (b + 2) % NBUF
            for s_, d_ in gather_pairs(b):
                pltpu.make_async_copy(s_, d_, gsem[b]).wait()

            @pl.when(i >= 2)
            def _():
                for s_, d_ in scatter_pairs(nxt):
                    pltpu.make_async_copy(s_, d_, ssem[nxt]).wait()

            @pl.when(i + 1 < n_slots)
            def _():
                fire_gathers(i + 1, nxt)

            for k in range(subch):
                for j in range(ekw):
                    sl = pl.ds(LANES * j, LANES)
                    dstv[b][k, sl] = ed[b][pl.ds(k * ec + EK + LANES * j, LANES)]

            def scale(g, carry2):
                ch = g // ekw
                woff = ch * ec + 2 * EK + (g - ch * ekw) * LANES
                w16 = lax.bitcast_convert_type(
                    ed[b][pl.ds(woff, LANES)], jnp.float32)
                for k in range(LANES):
                    w = lax.gather(
                        w16, jnp.full((LANES, 1), k, jnp.int32), gdn,
                        slice_sizes=(1,),
                        mode=lax.GatherScatterMode.PROMISE_IN_BOUNDS)
                    e = g * LANES + k
                    for j in range(nslice):
                        sl = pl.ds(LANES * j, LANES)
                        rows[b][e, sl] = rows[b][e, sl] * w
                return carry2

            lax.fori_loop(0, eks // LANES, scale, 0)

            @pl.when(i + 2 < n_slots)
            def _():
                fire_ed(i + 2, n2)

            for s_, d_ in scatter_pairs(b):
                pltpu.async_copy(s_, d_, ssem[b], add=True)

        fire_ed(0, 0)
        fire_gathers(0, 0)
        fire_ed(1, 1)

        def tri(t, carry):
            for k in range(NBUF):
                half(t * NBUF + k, k)
            return carry

        lax.fori_loop(0, n_slots // NBUF, tri, 0)
        for i in (n_slots - 2, n_slots - 1):
            for s_, d_ in scatter_pairs(i % NBUF):
                pltpu.make_async_copy(s_, d_, ssem[i % NBUF]).wait()
        plsc.subcore_barrier()

        @pl.when(s < NS - 1)
        def _():
            pltpu.sync_copy(hacc.at[pl.ds(row0, rmain)],
                            out.at[pl.ds(c * n_nodes + row0, rmain)])

        @pl.when(s == NS - 1)
        def _():
            pltpu.sync_copy(hacc.at[pl.ds(row0, rlast)],
                            out.at[pl.ds(c * n_nodes + row0, rlast)])

    return spmm


def _fused_body(axa_ref, axb_ref, w1_ref, w3_ref, b1_ref, o_ref):
    ax = axa_ref[...] + axb_ref[...]
    h = jnp.maximum(
        jnp.dot(ax, w1_ref[...], preferred_element_type=jnp.float32)
        + b1_ref[...], 0.0)
    o_ref[...] = jnp.dot(h, w3_ref[0], preferred_element_type=jnp.float32)


def _tc_fused(axcat, W1, w3, b1r, n, bm):
    _, fx = axcat.shape
    d, hh = W1.shape
    _, _, f2 = w3.shape
    gi = n // bm
    return pl.pallas_call(
        _fused_body,
        grid=(gi, NC),
        in_specs=[
            pl.BlockSpec((bm, fx), lambda i, j: (i, 0)),
            pl.BlockSpec((bm, fx), lambda i, j, _g=gi: (_g + i, 0)),
            pl.BlockSpec((d, hh), lambda i, j: (0, 0)),
            pl.BlockSpec((1, hh, f2), lambda i, j: (j, 0, 0)),
            pl.BlockSpec((1, hh), lambda i, j: (0, 0)),
        ],
        out_specs=pl.BlockSpec((bm, f2), lambda i, j, _g=gi: (j * _g + i, 0)),
        out_shape=jax.ShapeDtypeStruct((NC * n, f2), jnp.float32),
    )(axcat, axcat, W1, w3, b1r)


def _final_body(za_ref, zb_ref, b2_ref, o_ref, *, n_cls):
    z = jnp.concatenate([za_ref[...], zb_ref[...]], axis=1) + b2_ref[...]
    col = lax.broadcasted_iota(jnp.int32, z.shape, 1)
    zm = jnp.where(col < n_cls, z, jnp.float32(-1e30))
    m = jnp.max(zm, axis=1, keepdims=True)
    lse = jnp.log(jnp.sum(jnp.exp(zm - m), axis=1, keepdims=True))
    o_ref[...] = (z - m - lse)[:, :n_cls]


def _tc_final(ocat, b2r, n, n_cls, bm):
    _, f2 = ocat.shape
    cp = b2r.shape[1]
    gi = n // bm
    return pl.pallas_call(
        functools.partial(_final_body, n_cls=n_cls),
        grid=(gi,),
        in_specs=[
            pl.BlockSpec((bm, f2), lambda i: (i, 0)),
            pl.BlockSpec((bm, f2), lambda i, _g=gi: (_g + i, 0)),
            pl.BlockSpec((1, cp), lambda i: (0, 0)),
        ],
        out_specs=pl.BlockSpec((bm, n_cls), lambda i: (i, 0)),
        out_shape=jax.ShapeDtypeStruct((n, n_cls), jnp.float32),
    )(ocat, ocat, b2r)


def kernel(x, edge_index, edge_weight, W1, b1, W2, b2):
    n, d = x.shape
    h = W1.shape[1]
    n_cls = W2.shape[1]
    e = edge_weight.shape[0]
    bm = 1000
    cp = 64

    sub2 = 4
    quantum = NS * EK * NBUF * sub2
    e_pad = math.ceil(e / quantum) * quantum
    if e_pad != e:
        fill = jnp.arange(e_pad - e, dtype=jnp.int32) % n
        src = jnp.concatenate([edge_index[0], fill])
        dst = jnp.concatenate([edge_index[1], fill])
        ew = jnp.concatenate(
            [edge_weight, jnp.zeros((e_pad - e,), jnp.float32)])
    else:
        src, dst, ew = edge_index[0], edge_index[1], edge_weight
    n_chunks = e_pad // (NS * EK)
    edata = jnp.stack([
        src.reshape(NS, n_chunks, EK),
        dst.reshape(NS, n_chunks, EK),
        lax.bitcast_convert_type(ew, jnp.int32).reshape(NS, n_chunks, EK),
    ], axis=2).reshape(-1)

    W2p = jnp.pad(W2, ((0, 0), (0, cp - n_cls)))
    f2 = cp // NC
    w3 = jnp.stack([W2p[:, k * f2:(k + 1) * f2] for k in range(NC)])
    b2r = jnp.pad(b2, (0, cp - n_cls)).reshape(1, cp)
    b1r = b1.reshape(1, h)

    axcat = _make_spmm(n, d, e_pad, 1, "edge")(x, edata)
    s2cat = _tc_fused(axcat, W1, w3, b1r, n, bm)
    ocat = _make_spmm(n, cp // NC, e_pad, sub2)(s2cat, edata)
    return _tc_final(ocat, b2r, n, n_cls, bm)


# speedup vs baseline: 1.8702x; 1.0185x over previous
import functools
import math

import jax
import jax.numpy as jnp
from jax import lax
from jax.experimental import pallas as pl
from jax.experimental.pallas import tpu as pltpu
from jax.experimental.pallas import tpu_sc as plsc

NC = 2
NS = 16
LANES = 16
EK = 112


NBUF = 3


@functools.cache
def _make_spmm(n_nodes: int, feat: int, e_pad: int, subch: int,
               split: str = "col"):
    ekw = EK // LANES
    eks = EK * subch
    edges_per_sub = e_pad // NS
    n_slots = edges_per_sub // eks
    if split == "edge":
        assert n_slots % NC == 0
        n_slots //= NC
    assert edges_per_sub % eks == 0 and n_slots % NBUF == 0 and n_slots >= 6
    rmain = ((n_nodes // NS) + 7) // 8 * 8
    rlast = n_nodes - (NS - 1) * rmain
    assert 0 < rlast <= rmain and rlast % 8 == 0
    zr = 8
    nslice = feat // LANES
    ec = 3 * EK
    ecs = ec * subch

    mesh = plsc.VectorSubcoreMesh(core_axis_name="c", subcore_axis_name="s")

    @functools.partial(
        pl.kernel,
        mesh=mesh,
        compiler_params=pltpu.CompilerParams(
            use_tc_tiling_on_sc=(feat % 128 == 0)),
        out_type=jax.ShapeDtypeStruct((2 * n_nodes, feat), jnp.float32),
        scratch_types=(
            [pltpu.VMEM((ecs,), jnp.int32)] * NBUF
            + [pltpu.VMEM((subch, EK), jnp.int32)] * NBUF
            + [pltpu.VMEM((subch, EK), jnp.int32)] * NBUF
            + [pltpu.VMEM((eks, feat), jnp.float32)] * NBUF
            + [pltpu.VMEM_SHARED((n_nodes, feat), jnp.float32)]
            + [pltpu.SemaphoreType.DMA] * (3 * NBUF)
        ),
    )
    def spmm(sup, edata, out, *refs):
        ed = refs[0:NBUF]
        srcv = refs[NBUF:2 * NBUF]
        dstv = refs[2 * NBUF:3 * NBUF]
        rows = refs[3 * NBUF:4 * NBUF]
        hacc = refs[4 * NBUF]
        gsem = refs[4 * NBUF + 1:4 * NBUF + 1 + NBUF]
        ssem = refs[4 * NBUF + 1 + NBUF:4 * NBUF + 1 + 2 * NBUF]
        esem = refs[4 * NBUF + 1 + 2 * NBUF:]
        c = lax.axis_index("c")
        s = lax.axis_index("s")

        zeros16 = jnp.zeros((LANES,), jnp.float32)

        def zfill(i, carry):
            for j in range(nslice):
                rows[0][i, pl.ds(LANES * j, LANES)] = zeros16
            return carry

        lax.fori_loop(0, eks, zfill, 0)
        row0 = s * rmain

        def zcopies(total):
            off = 0
            while off < total:
                sz = min(eks, total - off)
                pltpu.sync_copy(rows[0].at[pl.ds(0, sz)],
                                hacc.at[pl.ds(row0 + off, sz)])
                off += sz

        @pl.when(s < NS - 1)
        def _():
            zcopies(rmain)

        @pl.when(s == NS - 1)
        def _():
            zcopies(rlast)

        plsc.subcore_barrier()

        if split == "edge":
            ebase = (s * NC + c) * n_slots * ecs
            coff = jnp.full((LANES,), 0, jnp.int32)
        else:
            ebase = s * n_slots * ecs
            coff = jnp.full((LANES,), c * n_nodes, jnp.int32)
        gdn = lax.GatherDimensionNumbers(
            offset_dims=(), collapsed_slice_dims=(0,), start_index_map=(0,))

        def fire_ed(i, b):
            pltpu.async_copy(
                edata.at[pl.ds(ebase + i * ecs, ecs)], ed[b], esem[b])

        def drain_ed(i, b):
            pltpu.make_async_copy(
                edata.at[pl.ds(ebase + i * ecs, ecs)], ed[b], esem[b]).wait()

        def gather_pairs(b):
            return [(sup.at[srcv[b].at[k]], rows[b].at[pl.ds(k * EK, EK)])
                    for k in range(subch)]

        def scatter_pairs(b):
            return [(rows[b].at[pl.ds(k * EK, EK)], hacc.at[dstv[b].at[k]])
                    for k in range(subch)]

        def fire_gathers(i, b):
            drain_ed(i, b)
            for k in range(subch):
                for j in range(ekw):
                    sl = pl.ds(LANES * j, LANES)
                    srcv[b][k, sl] = ed[b][pl.ds(k * ec + LANES * j, LANES)] + coff
            for s_, d_ in gather_pairs(b):
                pltpu.async_copy(s_, d_, gsem[b])

        def half(i, b):
            nxt = (b + 1) % NBUF
            n2 = (b + 2) % NBUF
            for s_, d_ in gather_pairs(b):
                pltpu.make_async_copy(s_, d_, gsem[b]).wait()

            @pl.when(i >= 2)
            def _():
                for s_, d_ in scatter_pairs(nxt):
                    pltpu.make_async_copy(s_, d_, ssem[nxt]).wait()

            @pl.when(i + 1 < n_slots)
            def _():
                fire_gathers(i + 1, nxt)

            for k in range(subch):
                for j in range(ekw):
                    sl = pl.ds(LANES * j, LANES)
                    dstv[b][k, sl] = ed[b][pl.ds(k * ec + EK + LANES * j, LANES)]

            def scale(g, carry2):
                ch = g // ekw
                woff = ch * ec + 2 * EK + (g - ch * ekw) * LANES
                w16 = lax.bitcast_convert_type(
                    ed[b][pl.ds(woff, LANES)], jnp.float32)
                for k in range(LANES):
                    w = lax.gather(
                        w16, jnp.full((LANES, 1), k, jnp.int32), gdn,
                        slice_sizes=(1,),
                        mode=lax.GatherScatterMode.PROMISE_IN_BOUNDS)
                    e = g * LANES + k
                    for j in range(nslice):
                        sl = pl.ds(LANES * j, LANES)
                        rows[b][e, sl] = rows[b][e, sl] * w
                return carry2

            lax.fori_loop(0, eks // LANES, scale, 0)

            @pl.when(i + 2 < n_slots)
            def _():
                fire_ed(i + 2, n2)

            for s_, d_ in scatter_pairs(b):
                pltpu.async_copy(s_, d_, ssem[b], add=True)

        fire_ed(0, 0)
        fire_gathers(0, 0)
        fire_ed(1, 1)

        def tri(t, carry):
            for k in range(NBUF):
                half(t * NBUF + k, k)
            return carry

        lax.fori_loop(0, n_slots // NBUF, tri, 0)
        for i in (n_slots - 2, n_slots - 1):
            for s_, d_ in scatter_pairs(i % NBUF):
                pltpu.make_async_copy(s_, d_, ssem[i % NBUF]).wait()
        plsc.subcore_barrier()

        @pl.when(s < NS - 1)
        def _():
            pltpu.sync_copy(hacc.at[pl.ds(row0, rmain)],
                            out.at[pl.ds(c * n_nodes + row0, rmain)])

        @pl.when(s == NS - 1)
        def _():
            pltpu.sync_copy(hacc.at[pl.ds(row0, rlast)],
                            out.at[pl.ds(c * n_nodes + row0, rlast)])

    return spmm


def _fused_body(axa_ref, axb_ref, w1_ref, w3_ref, b1_ref, o_ref):
    ax = axa_ref[...] + axb_ref[...]
    h = jnp.maximum(
        jnp.dot(ax, w1_ref[...], preferred_element_type=jnp.float32)
        + b1_ref[...], 0.0)
    o_ref[...] = jnp.dot(h, w3_ref[0], preferred_element_type=jnp.float32)


def _tc_fused(axcat, W1, w3, b1r, n, bm):
    _, fx = axcat.shape
    d, hh = W1.shape
    _, _, f2 = w3.shape
    gi = n // bm
    return pl.pallas_call(
        _fused_body,
        grid=(gi, NC),
        in_specs=[
            pl.BlockSpec((bm, fx), lambda i, j: (i, 0)),
            pl.BlockSpec((bm, fx), lambda i, j, _g=gi: (_g + i, 0)),
            pl.BlockSpec((d, hh), lambda i, j: (0, 0)),
            pl.BlockSpec((1, hh, f2), lambda i, j: (j, 0, 0)),
            pl.BlockSpec((1, hh), lambda i, j: (0, 0)),
        ],
        out_specs=pl.BlockSpec((bm, f2), lambda i, j, _g=gi: (j * _g + i, 0)),
        out_shape=jax.ShapeDtypeStruct((NC * n, f2), jnp.float32),
    )(axcat, axcat, W1, w3, b1r)


def _final_body(za_ref, zb_ref, b2_ref, o_ref, *, n_cls):
    z = jnp.concatenate([za_ref[...], zb_ref[...]], axis=1) + b2_ref[...]
    col = lax.broadcasted_iota(jnp.int32, z.shape, 1)
    zm = jnp.where(col < n_cls, z, jnp.float32(-1e30))
    m = jnp.max(zm, axis=1, keepdims=True)
    lse = jnp.log(jnp.sum(jnp.exp(zm - m), axis=1, keepdims=True))
    o_ref[...] = (z - m - lse)[:, :n_cls]


def _tc_final(ocat, b2r, n, n_cls, bm):
    _, f2 = ocat.shape
    cp = b2r.shape[1]
    gi = n // bm
    return pl.pallas_call(
        functools.partial(_final_body, n_cls=n_cls),
        grid=(gi,),
        in_specs=[
            pl.BlockSpec((bm, f2), lambda i: (i, 0)),
            pl.BlockSpec((bm, f2), lambda i, _g=gi: (_g + i, 0)),
            pl.BlockSpec((1, cp), lambda i: (0, 0)),
        ],
        out_specs=pl.BlockSpec((bm, n_cls), lambda i: (i, 0)),
        out_shape=jax.ShapeDtypeStruct((n, n_cls), jnp.float32),
    )(ocat, ocat, b2r)


def kernel(x, edge_index, edge_weight, W1, b1, W2, b2):
    n, d = x.shape
    h = W1.shape[1]
    n_cls = W2.shape[1]
    e = edge_weight.shape[0]
    bm = 1000
    cp = 64

    sub2 = 4
    quantum = NS * EK * NBUF * sub2
    e_pad = math.ceil(e / quantum) * quantum
    if e_pad != e:
        fill = jnp.arange(e_pad - e, dtype=jnp.int32) % n
        src = jnp.concatenate([edge_index[0], fill])
        dst = jnp.concatenate([edge_index[1], fill])
        ew = jnp.concatenate(
            [edge_weight, jnp.zeros((e_pad - e,), jnp.float32)])
    else:
        src, dst, ew = edge_index[0], edge_index[1], edge_weight
    n_chunks = e_pad // (NS * EK)
    edata = jnp.stack([
        src.reshape(NS, n_chunks, EK),
        dst.reshape(NS, n_chunks, EK),
        lax.bitcast_convert_type(ew, jnp.int32).reshape(NS, n_chunks, EK),
    ], axis=2).reshape(-1)

    W2p = jnp.pad(W2, ((0, 0), (0, cp - n_cls)))
    f2 = cp // NC
    w3 = jnp.stack([W2p[:, k * f2:(k + 1) * f2] for k in range(NC)])
    b2r = jnp.pad(b2, (0, cp - n_cls)).reshape(1, cp)
    b1r = b1.reshape(1, h)

    axcat = _make_spmm(n, d, e_pad, 1, "edge")(x, edata)
    s2cat = _tc_fused(axcat, W1, w3, b1r, n, bm)
    ocat = _make_spmm(n, cp // NC, e_pad, sub2)(s2cat, edata)
    return _tc_final(ocat, b2r, n, n_cls, bm)
